# baseline (speedup 1.0000x reference)
import functools

import jax
import jax.numpy as jnp
from jax import lax
from jax.experimental import pallas as pl
from jax.experimental.pallas import tpu as pltpu
from jax.experimental.pallas import tpu_sc as plsc

_UNROLL = 8
_NCHUNK = 8


def _quantize_body(chunk, K, x_hbm, c_hbm, o_hbm,
                   xv0, xv1, ov0, ov1, cv, isem0, isem1, osem0, osem1):
    nc = lax.axis_size("c")
    wid = lax.axis_index("s") * nc + lax.axis_index("c")
    base = wid * (chunk * _NCHUNK)
    pltpu.sync_copy(c_hbm, cv)

    cvec = cv[pl.ds(0, 16)]
    ctop = cv[pl.ds(K - 16, 16)]
    c0 = jnp.full((16,), cvec[0], jnp.float32)
    cK = jnp.full((16,), ctop[15], jnp.float32)
    inv_step = jnp.full((16,), K - 1, jnp.float32) / (cK - c0)
    bias = 0.5 - c0 * inv_step
    lo = jnp.full((16,), 0.5, jnp.float32)
    hi = jnp.full((16,), K - 0.5, jnp.float32)

    xbufs = (xv0, xv1)
    obufs = (ov0, ov1)
    isems = (isem0, isem1)
    osems = (osem0, osem1)

    def compute(xv, ov):
        @plsc.parallel_loop(0, chunk, step=16, unroll=_UNROLL)
        def _vec(off):
            v = xv[pl.ds(off, 16)]
            u = jnp.minimum(jnp.maximum(v * inv_step + bias, lo), hi)
            ov[pl.ds(off, 16)] = plsc.load_gather(cv, [u.astype(jnp.int32)])

    in_h = {}
    out_h = {}
    in_h[0] = pltpu.async_copy(x_hbm.at[pl.ds(base, chunk)], xbufs[0], isems[0])
    for g in range(_NCHUNK):
        in_h[g].wait()
        if g + 1 < _NCHUNK:
            in_h[g + 1] = pltpu.async_copy(
                x_hbm.at[pl.ds(base + (g + 1) * chunk, chunk)],
                xbufs[(g + 1) % 2], isems[(g + 1) % 2])
        if g >= 2:
            out_h[g - 2].wait()
        compute(xbufs[g % 2], obufs[g % 2])
        out_h[g] = pltpu.async_copy(
            obufs[g % 2], o_hbm.at[pl.ds(base + g * chunk, chunk)],
            osems[g % 2])
    for g in range(max(0, _NCHUNK - 2), _NCHUNK):
        out_h[g].wait()


def kernel(x, center):
    K = center.shape[0]
    n = x.size
    info = plsc.get_sparse_core_info()
    nw = info.num_cores * info.num_subcores
    grain = 16 * _UNROLL * _NCHUNK
    n_pad = ((n + nw * grain - 1) // (nw * grain)) * (nw * grain)
    chunk = n_pad // (nw * _NCHUNK)

    if x.ndim == 4 and x.shape[0] % 8 == 0 and x.shape[1] % 128 == 0:
        d0, d1, d2, d3 = x.shape
        flat = (
            x.transpose(2, 3, 0, 1)
            .reshape(d2, d3, d0 // 8, 8, d1 // 128, 128)
            .transpose(0, 1, 2, 4, 3, 5)
            .reshape(-1)
        )
        unscramble = lambda o: (
            o.reshape(d2, d3, d0 // 8, d1 // 128, 8, 128)
            .transpose(0, 1, 2, 4, 3, 5)
            .reshape(d2, d3, d0, d1)
            .transpose(2, 3, 0, 1)
        )
    else:
        flat = x.reshape(-1)
        unscramble = lambda o: o.reshape(x.shape)
    if n_pad != n:
        flat = jnp.pad(flat, (0, n_pad - n))

    mesh = plsc.VectorSubcoreMesh(core_axis_name="c", subcore_axis_name="s")
    body = functools.partial(_quantize_body, chunk, K)
    out = pl.kernel(
        body,
        out_type=jax.ShapeDtypeStruct((n_pad,), jnp.float32),
        mesh=mesh,
        scratch_types=[
            pltpu.VMEM((chunk,), jnp.float32),
            pltpu.VMEM((chunk,), jnp.float32),
            pltpu.VMEM((chunk,), jnp.float32),
            pltpu.VMEM((chunk,), jnp.float32),
            pltpu.VMEM((K,), jnp.float32),
            pltpu.SemaphoreType.DMA,
            pltpu.SemaphoreType.DMA,
            pltpu.SemaphoreType.DMA,
            pltpu.SemaphoreType.DMA,
        ],
        compiler_params=pltpu.CompilerParams(needs_layout_passes=False),
    )(flat, center)
    return unscramble(out[:n])


# speedup vs baseline: 1.3660x; 1.3660x over previous
import functools

import jax
import jax.numpy as jnp
from jax import lax
from jax.experimental import pallas as pl
from jax.experimental.pallas import tpu as pltpu
from jax.experimental.pallas import tpu_sc as plsc

_UNROLL = 8
_NCHUNK = 2


def _quantize_body(chunk, K, x_hbm, c_hbm, o_hbm,
                   xv0, xv1, ov0, ov1, cv, isem0, isem1, osem0, osem1):
    nc = lax.axis_size("c")
    wid = lax.axis_index("s") * nc + lax.axis_index("c")
    base = wid * (chunk * _NCHUNK)
    pltpu.sync_copy(c_hbm, cv)

    cvec = cv[pl.ds(0, 16)]
    ctop = cv[pl.ds(K - 16, 16)]
    c0 = jnp.full((16,), cvec[0], jnp.float32)
    cK = jnp.full((16,), ctop[15], jnp.float32)
    inv_step = jnp.full((16,), K - 1, jnp.float32) / (cK - c0)
    bias = 0.5 - c0 * inv_step
    lo = jnp.full((16,), 0.5, jnp.float32)
    hi = jnp.full((16,), K - 0.5, jnp.float32)

    xbufs = (xv0, xv1)
    obufs = (ov0, ov1)
    isems = (isem0, isem1)
    osems = (osem0, osem1)

    def compute(xv, ov):
        @plsc.parallel_loop(0, chunk, step=16, unroll=_UNROLL)
        def _vec(off):
            v = xv[pl.ds(off, 16)]
            u = jnp.minimum(jnp.maximum(v * inv_step + bias, lo), hi)
            ov[pl.ds(off, 16)] = plsc.load_gather(cv, [u.astype(jnp.int32)])

    in_h = {}
    out_h = {}
    in_h[0] = pltpu.async_copy(x_hbm.at[pl.ds(base, chunk)], xbufs[0], isems[0])
    for g in range(_NCHUNK):
        in_h[g].wait()
        if g + 1 < _NCHUNK:
            in_h[g + 1] = pltpu.async_copy(
                x_hbm.at[pl.ds(base + (g + 1) * chunk, chunk)],
                xbufs[(g + 1) % 2], isems[(g + 1) % 2])
        if g >= 2:
            out_h[g - 2].wait()
        compute(xbufs[g % 2], obufs[g % 2])
        out_h[g] = pltpu.async_copy(
            obufs[g % 2], o_hbm.at[pl.ds(base + g * chunk, chunk)],
            osems[g % 2])
    for g in range(max(0, _NCHUNK - 2), _NCHUNK):
        out_h[g].wait()


def kernel(x, center):
    K = center.shape[0]
    n = x.size
    info = plsc.get_sparse_core_info()
    nw = info.num_cores * info.num_subcores
    grain = 16 * _UNROLL * _NCHUNK
    n_pad = ((n + nw * grain - 1) // (nw * grain)) * (nw * grain)
    chunk = n_pad // (nw * _NCHUNK)

    if x.ndim == 4 and x.shape[0] % 8 == 0 and x.shape[1] % 128 == 0:
        d0, d1, d2, d3 = x.shape
        flat = (
            x.transpose(2, 3, 0, 1)
            .reshape(d2, d3, d0 // 8, 8, d1 // 128, 128)
            .transpose(0, 1, 2, 4, 3, 5)
            .reshape(-1)
        )
        unscramble = lambda o: (
            o.reshape(d2, d3, d0 // 8, d1 // 128, 8, 128)
            .transpose(0, 1, 2, 4, 3, 5)
            .reshape(d2, d3, d0, d1)
            .transpose(2, 3, 0, 1)
        )
    else:
        flat = x.reshape(-1)
        unscramble = lambda o: o.reshape(x.shape)
    if n_pad != n:
        flat = jnp.pad(flat, (0, n_pad - n))

    mesh = plsc.VectorSubcoreMesh(core_axis_name="c", subcore_axis_name="s")
    body = functools.partial(_quantize_body, chunk, K)
    out = pl.kernel(
        body,
        out_type=jax.ShapeDtypeStruct((n_pad,), jnp.float32),
        mesh=mesh,
        scratch_types=[
            pltpu.VMEM((chunk,), jnp.float32),
            pltpu.VMEM((chunk,), jnp.float32),
            pltpu.VMEM((chunk,), jnp.float32),
            pltpu.VMEM((chunk,), jnp.float32),
            pltpu.VMEM((K,), jnp.float32),
            pltpu.SemaphoreType.DMA,
            pltpu.SemaphoreType.DMA,
            pltpu.SemaphoreType.DMA,
            pltpu.SemaphoreType.DMA,
        ],
        compiler_params=pltpu.CompilerParams(needs_layout_passes=False),
    )(flat, center)
    return unscramble(out[:n])


# speedup vs baseline: 1.3693x; 1.0025x over previous
import functools

import jax
import jax.numpy as jnp
from jax import lax
from jax.experimental import pallas as pl
from jax.experimental.pallas import tpu as pltpu
from jax.experimental.pallas import tpu_sc as plsc

_UNROLL = 16
_NCHUNK = 2


def _quantize_body(chunk, K, x_hbm, c_hbm, o_hbm,
                   xv0, xv1, ov0, ov1, cv, isem0, isem1, osem0, osem1):
    nc = lax.axis_size("c")
    wid = lax.axis_index("s") * nc + lax.axis_index("c")
    base = wid * (chunk * _NCHUNK)
    pltpu.sync_copy(c_hbm, cv)

    cvec = cv[pl.ds(0, 16)]
    ctop = cv[pl.ds(K - 16, 16)]
    c0 = jnp.full((16,), cvec[0], jnp.float32)
    cK = jnp.full((16,), ctop[15], jnp.float32)
    inv_step = jnp.full((16,), K - 1, jnp.float32) / (cK - c0)
    bias = 0.5 - c0 * inv_step
    lo = jnp.full((16,), 0.5, jnp.float32)
    hi = jnp.full((16,), K - 0.5, jnp.float32)

    xbufs = (xv0, xv1)
    obufs = (ov0, ov1)
    isems = (isem0, isem1)
    osems = (osem0, osem1)

    def compute(xv, ov):
        @plsc.parallel_loop(0, chunk, step=16, unroll=_UNROLL)
        def _vec(off):
            v = xv[pl.ds(off, 16)]
            u = jnp.minimum(jnp.maximum(v * inv_step + bias, lo), hi)
            ov[pl.ds(off, 16)] = plsc.load_gather(cv, [u.astype(jnp.int32)])

    in_h = {}
    out_h = {}
    in_h[0] = pltpu.async_copy(x_hbm.at[pl.ds(base, chunk)], xbufs[0], isems[0])
    for g in range(_NCHUNK):
        in_h[g].wait()
        if g + 1 < _NCHUNK:
            in_h[g + 1] = pltpu.async_copy(
                x_hbm.at[pl.ds(base + (g + 1) * chunk, chunk)],
                xbufs[(g + 1) % 2], isems[(g + 1) % 2])
        if g >= 2:
            out_h[g - 2].wait()
        compute(xbufs[g % 2], obufs[g % 2])
        out_h[g] = pltpu.async_copy(
            obufs[g % 2], o_hbm.at[pl.ds(base + g * chunk, chunk)],
            osems[g % 2])
    for g in range(max(0, _NCHUNK - 2), _NCHUNK):
        out_h[g].wait()


def kernel(x, center):
    K = center.shape[0]
    n = x.size
    info = plsc.get_sparse_core_info()
    nw = info.num_cores * info.num_subcores
    grain = 16 * _UNROLL * _NCHUNK
    n_pad = ((n + nw * grain - 1) // (nw * grain)) * (nw * grain)
    chunk = n_pad // (nw * _NCHUNK)

    if x.ndim == 4 and x.shape[0] % 8 == 0 and x.shape[1] % 128 == 0:
        d0, d1, d2, d3 = x.shape
        flat = (
            x.transpose(2, 3, 0, 1)
            .reshape(d2, d3, d0 // 8, 8, d1 // 128, 128)
            .transpose(0, 1, 2, 4, 3, 5)
            .reshape(-1)
        )
        unscramble = lambda o: (
            o.reshape(d2, d3, d0 // 8, d1 // 128, 8, 128)
            .transpose(0, 1, 2, 4, 3, 5)
            .reshape(d2, d3, d0, d1)
            .transpose(2, 3, 0, 1)
        )
    else:
        flat = x.reshape(-1)
        unscramble = lambda o: o.reshape(x.shape)
    if n_pad != n:
        flat = jnp.pad(flat, (0, n_pad - n))

    mesh = plsc.VectorSubcoreMesh(core_axis_name="c", subcore_axis_name="s")
    body = functools.partial(_quantize_body, chunk, K)
    out = pl.kernel(
        body,
        out_type=jax.ShapeDtypeStruct((n_pad,), jnp.float32),
        mesh=mesh,
        scratch_types=[
            pltpu.VMEM((chunk,), jnp.float32),
            pltpu.VMEM((chunk,), jnp.float32),
            pltpu.VMEM((chunk,), jnp.float32),
            pltpu.VMEM((chunk,), jnp.float32),
            pltpu.VMEM((K,), jnp.float32),
            pltpu.SemaphoreType.DMA,
            pltpu.SemaphoreType.DMA,
            pltpu.SemaphoreType.DMA,
            pltpu.SemaphoreType.DMA,
        ],
        compiler_params=pltpu.CompilerParams(needs_layout_passes=False),
    )(flat, center)
    return unscramble(out[:n])
